# baseline (speedup 1.0000x reference)
import functools

import jax
import jax.numpy as jnp
from jax import lax
from jax.experimental import pallas as pl
from jax.experimental.pallas import tpu as pltpu
from jax.experimental.pallas import tpu_sc as plsc

NUM_STOCKS = 100000
CELL_SIZE = 64
BATCH = 16384
ROWS = NUM_STOCKS + 1

NC, NS = 2, 16
NW = NC * NS
B_PER_W = BATCH // NW
CHUNK = 128
NCHUNK = B_PER_W // CHUNK

PACK_BR = 2048


def _pack_tables(emb0, emb1):
    def body(a_ref, b_ref, o_ref):
        o_ref[:, :CELL_SIZE] = a_ref[...]
        o_ref[:, CELL_SIZE:] = b_ref[...]

    return pl.pallas_call(
        body,
        grid=(pl.cdiv(ROWS, PACK_BR),),
        in_specs=[
            pl.BlockSpec((PACK_BR, CELL_SIZE), lambda i: (i, 0)),
            pl.BlockSpec((PACK_BR, CELL_SIZE), lambda i: (i, 0)),
        ],
        out_specs=pl.BlockSpec((PACK_BR, 2 * CELL_SIZE), lambda i: (i, 0)),
        out_shape=jax.ShapeDtypeStruct((ROWS, 2 * CELL_SIZE), jnp.float32),
    )(emb0, emb1)


def _encoder_gather(idx_flat, packed):
    mesh = plsc.VectorSubcoreMesh(core_axis_name="c", subcore_axis_name="s")
    out_t = jax.ShapeDtypeStruct((BATCH, 2 * CELL_SIZE), jnp.float32)

    @functools.partial(
        pl.kernel,
        out_type=out_t,
        mesh=mesh,
        scratch_types=[
            pltpu.VMEM((B_PER_W,), jnp.int32),
            pltpu.VMEM((B_PER_W, 2 * CELL_SIZE), jnp.float32),
            pltpu.SemaphoreType.DMA,
            pltpu.SemaphoreType.DMA,
        ],
    )
    def k(tab_hbm, idx_hbm, o_hbm, idx_v, rows_v, sem_g, sem_w):
        wid = lax.axis_index("s") * NC + lax.axis_index("c")
        base = wid * B_PER_W
        pltpu.sync_copy(idx_hbm.at[pl.ds(base, B_PER_W)], idx_v)

        gathers = []
        for j in range(NCHUNK):
            sl = pl.ds(j * CHUNK, CHUNK)
            gathers.append(pltpu.async_copy(
                tab_hbm.at[idx_v.at[sl]], rows_v.at[sl], sem_g))
        for c in gathers:
            c.wait()
        pltpu.async_copy(rows_v, o_hbm.at[pl.ds(base, B_PER_W)], sem_w).wait()

    return k(packed, idx_flat)


def kernel(Stock_ID, emb0, emb1):
    idx_flat = Stock_ID.reshape(BATCH).astype(jnp.int32)
    packed = _pack_tables(emb0, emb1)
    out = _encoder_gather(idx_flat, packed)
    return (out[:, :CELL_SIZE], out[:, CELL_SIZE:])


# speedup vs baseline: 1.1240x; 1.1240x over previous
import functools

import jax
import jax.numpy as jnp
from jax import lax
from jax.experimental import pallas as pl
from jax.experimental.pallas import tpu as pltpu
from jax.experimental.pallas import tpu_sc as plsc

NUM_STOCKS = 100000
CELL_SIZE = 64
BATCH = 16384

NC, NS = 2, 16
NW = NC * NS
B_PER_W = BATCH // NW
W_PER_TILE = B_PER_W * CELL_SIZE


def _gather_one(idx_flat, e_flat):
    mesh = plsc.VectorSubcoreMesh(core_axis_name="c", subcore_axis_name="s")
    out_t = jax.ShapeDtypeStruct((BATCH * CELL_SIZE,), jnp.float32)

    @functools.partial(
        pl.kernel,
        out_type=out_t,
        mesh=mesh,
        scratch_types=[
            pltpu.VMEM((B_PER_W,), jnp.int32),
            pltpu.VMEM((W_PER_TILE,), jnp.float32),
            pltpu.SemaphoreType.DMA,
            pltpu.SemaphoreType.DMA,
        ],
    )
    def k(e_hbm, idx_hbm, o_hbm, idx_v, rows_v, sem_g, sem_w):
        wid = lax.axis_index("s") * NC + lax.axis_index("c")
        base = wid * B_PER_W
        pltpu.sync_copy(idx_hbm.at[pl.ds(base, B_PER_W)], idx_v)

        @pl.loop(0, B_PER_W, step=16)
        def _(j):
            v = idx_v[pl.ds(j, 16)]
            for t in range(16):
                src = v[t] * CELL_SIZE
                dst = (j + t) * CELL_SIZE
                pltpu.make_async_copy(
                    e_hbm.at[pl.ds(src, CELL_SIZE)],
                    rows_v.at[pl.ds(dst, CELL_SIZE)],
                    sem_g).start()

        obase = base * CELL_SIZE
        pltpu.make_async_copy(
            o_hbm.at[pl.ds(obase, W_PER_TILE)], rows_v, sem_g).wait()
        pltpu.async_copy(
            rows_v, o_hbm.at[pl.ds(obase, W_PER_TILE)], sem_w).wait()

    return k(e_flat, idx_flat)


def kernel(Stock_ID, emb0, emb1):
    idx_flat = Stock_ID.reshape(BATCH).astype(jnp.int32)
    o0 = _gather_one(idx_flat, emb0.reshape(-1))
    o1 = _gather_one(idx_flat, emb1.reshape(-1))
    return (o0.reshape(BATCH, CELL_SIZE), o1.reshape(BATCH, CELL_SIZE))


# speedup vs baseline: 1.5570x; 1.3852x over previous
import functools

import jax
import jax.numpy as jnp
from jax import lax
from jax.experimental import pallas as pl
from jax.experimental.pallas import tpu as pltpu
from jax.experimental.pallas import tpu_sc as plsc

NUM_STOCKS = 100000
CELL_SIZE = 64
BATCH = 16384

NC, NS = 2, 16
NW = NC * NS
B_PER_W = BATCH // NW
P_ROWS = 256
N_PASS = B_PER_W // P_ROWS


def _encoder_gather(idx_flat, emb0, emb1):
    mesh = plsc.VectorSubcoreMesh(core_axis_name="c", subcore_axis_name="s")
    out_t = (
        jax.ShapeDtypeStruct((BATCH, CELL_SIZE), jnp.float32),
        jax.ShapeDtypeStruct((BATCH, CELL_SIZE), jnp.float32),
    )

    @functools.partial(
        pl.kernel,
        out_type=out_t,
        mesh=mesh,
        scratch_types=[
            pltpu.VMEM((B_PER_W,), jnp.int32),
            pltpu.VMEM((P_ROWS, CELL_SIZE), jnp.float32),
            pltpu.VMEM((P_ROWS, CELL_SIZE), jnp.float32),
            pltpu.SemaphoreType.DMA,
            pltpu.SemaphoreType.DMA,
            pltpu.SemaphoreType.DMA,
            pltpu.SemaphoreType.DMA,
        ],
    )
    def k(e0_hbm, e1_hbm, idx_hbm, o0_hbm, o1_hbm,
          idx_v, rows0_v, rows1_v, sem_g0, sem_g1, sem_w0, sem_w1):
        wid = lax.axis_index("s") * NC + lax.axis_index("c")
        base = wid * B_PER_W
        pltpu.sync_copy(idx_hbm.at[pl.ds(base, B_PER_W)], idx_v)

        for p in range(N_PASS):
            @pl.loop(p * P_ROWS, (p + 1) * P_ROWS, step=16)
            def _(j):
                v = idx_v[pl.ds(j, 16)]
                for t in range(16):
                    r = v[t]
                    d = j + t - p * P_ROWS
                    pltpu.make_async_copy(
                        e0_hbm.at[r], rows0_v.at[d], sem_g0).start()
                    pltpu.make_async_copy(
                        e1_hbm.at[r], rows1_v.at[d], sem_g1).start()

            pbase = base + p * P_ROWS
            pltpu.make_async_copy(
                o0_hbm.at[pl.ds(pbase, P_ROWS)], rows0_v, sem_g0).wait()
            w0 = pltpu.async_copy(
                rows0_v, o0_hbm.at[pl.ds(pbase, P_ROWS)], sem_w0)
            pltpu.make_async_copy(
                o1_hbm.at[pl.ds(pbase, P_ROWS)], rows1_v, sem_g1).wait()
            w1 = pltpu.async_copy(
                rows1_v, o1_hbm.at[pl.ds(pbase, P_ROWS)], sem_w1)
            w0.wait()
            w1.wait()

    return k(emb0, emb1, idx_flat)


def kernel(Stock_ID, emb0, emb1):
    idx_flat = Stock_ID.reshape(BATCH).astype(jnp.int32)
    return _encoder_gather(idx_flat, emb0, emb1)
